# baseline (speedup 1.0000x reference)
import jax
import jax.numpy as jnp
from jax.experimental import pallas as pl
from jax.experimental.pallas import tpu as pltpu

N = 4096
D = 128
FC1 = 64
BM = 256
IB = N // BM
NBUF = 4
NSUB = 2
SUBR = BM // NSUB


def _gcn_kernel(V_ref, adj_hbm, w1_ref, w2_ref, w3_ref, bg_ref,
                fc0w_ref, fc0b_ref, fc1w_ref, fc1b_ref, out_ref,
                abuf, hcat_ref, m0_ref, m1_ref, sem):

    def copies(blk, slot):
        return [
            pltpu.make_async_copy(
                adj_hbm.at[pl.ds(blk * BM + q * SUBR, SUBR), :],
                abuf.at[slot, pl.ds(q * SUBR, SUBR), :],
                sem.at[slot, q])
            for q in range(NSUB)
        ]

    def build_masks(k, m_ref):
        slot = jax.lax.rem(k, NBUF)
        for c in copies(k, slot):
            c.wait()
        a = abuf[slot]
        m_ref[:, pl.ds(0, N)] = (a == 1).astype(jnp.bfloat16)
        m_ref[:, pl.ds(N, N)] = (a == 2).astype(jnp.bfloat16)
        m_ref[:, pl.ds(2 * N, N)] = (a == 3).astype(jnp.bfloat16)

        @pl.when(k + NBUF < IB)
        def _prefetch():
            for c in copies(k + NBUF, slot):
                c.start()

    for b in range(NBUF):
        for c in copies(b, b):
            c.start()

    vblk = V_ref[:, :]
    hcat_ref[pl.ds(0, N), :] = jnp.dot(
        vblk, w1_ref[:, :],
        preferred_element_type=jnp.float32).astype(jnp.bfloat16)
    hcat_ref[pl.ds(N, N), :] = jnp.dot(
        vblk, w2_ref[:, :],
        preferred_element_type=jnp.float32).astype(jnp.bfloat16)
    hcat_ref[pl.ds(2 * N, N), :] = jnp.dot(
        vblk, w3_ref[:, :],
        preferred_element_type=jnp.float32).astype(jnp.bfloat16)

    build_masks(0, m0_ref)

    def body(t, zacc):
        k0 = 2 * t
        o0 = jnp.dot(m0_ref[:, :], hcat_ref[:, :],
                     preferred_element_type=jnp.float32)
        build_masks(k0 + 1, m1_ref)
        z0 = jnp.maximum(o0 + bg_ref[:, :], 0.0)

        o1 = jnp.dot(m1_ref[:, :], hcat_ref[:, :],
                     preferred_element_type=jnp.float32)

        @pl.when(k0 + 2 < IB)
        def _build_next():
            build_masks(k0 + 2, m0_ref)

        z1 = jnp.maximum(o1 + bg_ref[:, :], 0.0)
        return (zacc + jnp.sum(z0, axis=0, keepdims=True)
                + jnp.sum(z1, axis=0, keepdims=True))

    zs = jax.lax.fori_loop(0, IB // 2, body, jnp.zeros((1, D), jnp.float32))

    h0 = jax.lax.dot_general(
        zs, fc0w_ref[:, :], (((1,), (1,)), ((), ())),
        preferred_element_type=jnp.float32) + fc0b_ref[:, :]
    h0 = jnp.maximum(h0, 0.0)
    y = jnp.sum(h0 * fc1w_ref[:, :] + fc1b_ref[:, :])
    out_ref[:, :] = jnp.full((1, 1), jax.nn.sigmoid(y), jnp.float32)


def kernel(V, adj, w1, w2, w3, bg, fc0_w, fc0_b, fc1_w, fc1_b):
    bg2 = bg.reshape(1, D)
    fc0b2 = fc0_b.reshape(1, FC1)
    fc1b2 = jnp.broadcast_to(fc1_b.reshape(1, 1) / FC1, (1, FC1))
    out = pl.pallas_call(
        _gcn_kernel,
        in_specs=[
            pl.BlockSpec((N, D), lambda: (0, 0)),
            pl.BlockSpec(memory_space=pl.ANY),
            pl.BlockSpec((D, D), lambda: (0, 0)),
            pl.BlockSpec((D, D), lambda: (0, 0)),
            pl.BlockSpec((D, D), lambda: (0, 0)),
            pl.BlockSpec((1, D), lambda: (0, 0)),
            pl.BlockSpec((FC1, D), lambda: (0, 0)),
            pl.BlockSpec((1, FC1), lambda: (0, 0)),
            pl.BlockSpec((1, FC1), lambda: (0, 0)),
            pl.BlockSpec((1, FC1), lambda: (0, 0)),
        ],
        out_specs=pl.BlockSpec((1, 1), lambda: (0, 0)),
        out_shape=jax.ShapeDtypeStruct((1, 1), jnp.float32),
        scratch_shapes=[
            pltpu.VMEM((NBUF, BM, N), jnp.int32),
            pltpu.VMEM((3 * N, D), jnp.bfloat16),
            pltpu.VMEM((BM, 3 * N), jnp.bfloat16),
            pltpu.VMEM((BM, 3 * N), jnp.bfloat16),
            pltpu.SemaphoreType.DMA((NBUF, NSUB)),
        ],
        compiler_params=pltpu.CompilerParams(
            vmem_limit_bytes=100 * 1024 * 1024),
    )(V, adj, w1, w2, w3, bg2, fc0_w, fc0b2, fc1_w, fc1b2)
    return out.reshape(1)


# speedup vs baseline: 1.2043x; 1.2043x over previous
import jax
import jax.numpy as jnp
from jax.experimental import pallas as pl
from jax.experimental.pallas import tpu as pltpu

N = 4096
D = 128
FC1 = 64
BM = 512
IB = N // BM
NBUF = 3
NSUB = 4
SUBR = BM // NSUB


def _gcn_kernel(V_ref, adj_hbm, w1_ref, w2_ref, w3_ref, bg_ref,
                fc0w_ref, fc0b_ref, fc1w_ref, fc1b_ref, out_ref,
                abuf, hcat_ref, sem):

    def copies(blk, slot):
        return [
            pltpu.make_async_copy(
                adj_hbm.at[pl.ds(blk * BM + q * SUBR, SUBR), :],
                abuf.at[slot, pl.ds(q * SUBR, SUBR), :],
                sem.at[slot, q])
            for q in range(NSUB)
        ]

    for b in range(NBUF):
        for c in copies(b, b):
            c.start()

    vblk = V_ref[:, :]
    hcat_ref[pl.ds(0, N), :] = jnp.dot(
        vblk, w1_ref[:, :],
        preferred_element_type=jnp.float32).astype(jnp.bfloat16)
    hcat_ref[pl.ds(N, N), :] = jnp.dot(
        vblk, w2_ref[:, :],
        preferred_element_type=jnp.float32).astype(jnp.bfloat16)
    hcat_ref[pl.ds(2 * N, N), :] = jnp.dot(
        vblk, w3_ref[:, :],
        preferred_element_type=jnp.float32).astype(jnp.bfloat16)

    def body(k, zacc):
        slot = jax.lax.rem(k, NBUF)
        for c in copies(k, slot):
            c.wait()
        a = abuf[slot]
        mcat = jnp.concatenate(
            [(a == 1).astype(jnp.bfloat16),
             (a == 2).astype(jnp.bfloat16),
             (a == 3).astype(jnp.bfloat16)], axis=1)
        o = jnp.dot(mcat, hcat_ref[:, :], preferred_element_type=jnp.float32)

        @pl.when(k + NBUF < IB)
        def _prefetch():
            for c in copies(k + NBUF, slot):
                c.start()

        z = jnp.maximum(o + bg_ref[:, :], 0.0)
        return zacc + jnp.sum(z, axis=0, keepdims=True)

    zs = jax.lax.fori_loop(0, IB, body, jnp.zeros((1, D), jnp.float32), unroll=2)

    h0 = jax.lax.dot_general(
        zs, fc0w_ref[:, :], (((1,), (1,)), ((), ())),
        preferred_element_type=jnp.float32) + fc0b_ref[:, :]
    h0 = jnp.maximum(h0, 0.0)
    y = jnp.sum(h0 * fc1w_ref[:, :] + fc1b_ref[:, :])
    out_ref[:, :] = jnp.full((1, 1), jax.nn.sigmoid(y), jnp.float32)


def kernel(V, adj, w1, w2, w3, bg, fc0_w, fc0_b, fc1_w, fc1_b):
    bg2 = bg.reshape(1, D)
    fc0b2 = fc0_b.reshape(1, FC1)
    fc1b2 = jnp.broadcast_to(fc1_b.reshape(1, 1) / FC1, (1, FC1))
    out = pl.pallas_call(
        _gcn_kernel,
        in_specs=[
            pl.BlockSpec((N, D), lambda: (0, 0)),
            pl.BlockSpec(memory_space=pl.ANY),
            pl.BlockSpec((D, D), lambda: (0, 0)),
            pl.BlockSpec((D, D), lambda: (0, 0)),
            pl.BlockSpec((D, D), lambda: (0, 0)),
            pl.BlockSpec((1, D), lambda: (0, 0)),
            pl.BlockSpec((FC1, D), lambda: (0, 0)),
            pl.BlockSpec((1, FC1), lambda: (0, 0)),
            pl.BlockSpec((1, FC1), lambda: (0, 0)),
            pl.BlockSpec((1, FC1), lambda: (0, 0)),
        ],
        out_specs=pl.BlockSpec((1, 1), lambda: (0, 0)),
        out_shape=jax.ShapeDtypeStruct((1, 1), jnp.float32),
        scratch_shapes=[
            pltpu.VMEM((NBUF, BM, N), jnp.int32),
            pltpu.VMEM((3 * N, D), jnp.bfloat16),
            pltpu.SemaphoreType.DMA((NBUF, NSUB)),
        ],
        compiler_params=pltpu.CompilerParams(
            vmem_limit_bytes=100 * 1024 * 1024),
    )(V, adj, w1, w2, w3, bg2, fc0_w, fc0b2, fc1_w, fc1b2)
    return out.reshape(1)


# speedup vs baseline: 2.0721x; 1.7206x over previous
import jax
import jax.numpy as jnp
from jax.experimental import pallas as pl
from jax.experimental.pallas import tpu as pltpu

N = 4096
D = 128
FC1 = 64
BM = 512
IB = N // BM
NBUF = 3
NSUB = 4
SUBR = BM // NSUB


def _gcn_kernel(V_ref, adj_hbm, w1_ref, w2_ref, w3_ref, bg_ref,
                fc0w_ref, fc0b_ref, fc1w_ref, fc1b_ref, out_ref,
                abuf, hcat_ref, sem):

    def copies(blk, slot):
        return [
            pltpu.make_async_copy(
                adj_hbm.at[pl.ds(blk * BM + q * SUBR, SUBR), :],
                abuf.at[slot, pl.ds(q * SUBR, SUBR), :],
                sem.at[slot, q])
            for q in range(NSUB)
        ]

    for b in range(NBUF):
        for c in copies(b, b):
            c.start()

    vblk = V_ref[:, :]
    hcat_ref[pl.ds(0, N), :] = jnp.dot(
        vblk, w1_ref[:, :],
        preferred_element_type=jnp.float32).astype(jnp.bfloat16)
    hcat_ref[pl.ds(N, N), :] = jnp.dot(
        vblk, w2_ref[:, :],
        preferred_element_type=jnp.float32).astype(jnp.bfloat16)
    hcat_ref[pl.ds(2 * N, N), :] = jnp.dot(
        vblk, w3_ref[:, :],
        preferred_element_type=jnp.float32).astype(jnp.bfloat16)

    def body(k, zacc):
        slot = jax.lax.rem(k, NBUF)
        for c in copies(k, slot):
            c.wait()
        a = abuf[slot]
        mcat = jnp.concatenate(
            [(a == 1).astype(jnp.bfloat16),
             (a == 2).astype(jnp.bfloat16),
             (a == 3).astype(jnp.bfloat16)], axis=1)
        o = jnp.dot(mcat, hcat_ref[:, :], preferred_element_type=jnp.float32)

        @pl.when(k + NBUF < IB)
        def _prefetch():
            for c in copies(k + NBUF, slot):
                c.start()

        z = jnp.maximum(o + bg_ref[:, :], 0.0)
        return zacc + jnp.sum(z, axis=0, keepdims=True)

    zs = jax.lax.fori_loop(0, IB, body, jnp.zeros((1, D), jnp.float32))

    h0 = jax.lax.dot_general(
        zs, fc0w_ref[:, :], (((1,), (1,)), ((), ())),
        preferred_element_type=jnp.float32) + fc0b_ref[:, :]
    h0 = jnp.maximum(h0, 0.0)
    y = jnp.sum(h0 * fc1w_ref[:, :] + fc1b_ref[:, :])
    out_ref[:, :] = jnp.full((1, 1), jax.nn.sigmoid(y), jnp.float32)


def kernel(V, adj, w1, w2, w3, bg, fc0_w, fc0_b, fc1_w, fc1_b):
    bg2 = bg.reshape(1, D)
    fc0b2 = fc0_b.reshape(1, FC1)
    fc1b2 = jnp.broadcast_to(fc1_b.reshape(1, 1) / FC1, (1, FC1))
    out = pl.pallas_call(
        _gcn_kernel,
        in_specs=[
            pl.BlockSpec((N, D), lambda: (0, 0)),
            pl.BlockSpec(memory_space=pl.ANY),
            pl.BlockSpec((D, D), lambda: (0, 0)),
            pl.BlockSpec((D, D), lambda: (0, 0)),
            pl.BlockSpec((D, D), lambda: (0, 0)),
            pl.BlockSpec((1, D), lambda: (0, 0)),
            pl.BlockSpec((FC1, D), lambda: (0, 0)),
            pl.BlockSpec((1, FC1), lambda: (0, 0)),
            pl.BlockSpec((1, FC1), lambda: (0, 0)),
            pl.BlockSpec((1, FC1), lambda: (0, 0)),
        ],
        out_specs=pl.BlockSpec((1, 1), lambda: (0, 0)),
        out_shape=jax.ShapeDtypeStruct((1, 1), jnp.float32),
        scratch_shapes=[
            pltpu.VMEM((NBUF, BM, N), jnp.int32),
            pltpu.VMEM((3 * N, D), jnp.bfloat16),
            pltpu.SemaphoreType.DMA((NBUF, NSUB)),
        ],
        compiler_params=pltpu.CompilerParams(
            vmem_limit_bytes=100 * 1024 * 1024),
    )(V, adj, w1, w2, w3, bg2, fc0_w, fc0b2, fc1_w, fc1b2)
    return out.reshape(1)
